# baseline (speedup 1.0000x reference)
import functools

import jax
import jax.numpy as jnp
from jax import lax
from jax.experimental import pallas as pl
from jax.experimental.pallas import tpu as pltpu
from jax.experimental.pallas import tpu_sc as plsc

NC, NS, L = 2, 16, 16
NW = NC * NS
ROWS, COLS = 128, 100000
RPW = ROWS // NW
CHUNK = 50000
CPR = COLS // CHUNK
NCH = RPW * CPR
VECS = CHUNK // L
UNROLL = 5
NEG = float("-inf")


def _sc_body(pred_hbm, true_hbm, out_hbm, true_v, buf0, buf1, part_v,
             sem0, sem1):
    c = lax.axis_index("c")
    s = lax.axis_index("s")
    wid = s * NC + c

    pltpu.sync_copy(true_hbm, true_v)

    def chunk_src(ch):
        return pred_hbm.at[wid * RPW + ch // CPR,
                           pl.ds((ch % CPR) * CHUNK, CHUNK)]

    bufs = (buf0, buf1)
    sems = (sem0, sem1)
    descs = [None] * NCH
    descs[0] = pltpu.async_copy(chunk_src(0), bufs[0], sems[0])

    lane = lax.iota(jnp.int32, L)
    total = jnp.float32(0.0)
    for r in range(RPW):
        row_idx = jnp.broadcast_to(wid * RPW + r, (L,)).astype(jnp.int32)
        t_r = plsc.load_gather(true_v, [row_idx])
        acc = jnp.full((L,), NEG, jnp.float32)
        tl = jnp.full((L,), NEG, jnp.float32)
        for h in range(CPR):
            ch = r * CPR + h
            b = ch % 2
            if ch + 1 < NCH:
                nb = (ch + 1) % 2
                descs[ch + 1] = pltpu.async_copy(chunk_src(ch + 1),
                                                 bufs[nb], sems[nb])
            descs[ch].wait()
            buf = bufs[b]

            p = t_r - h * CHUNK
            inr = (p >= 0) & (p < CHUNK)
            pc = jnp.clip(p, 0, CHUNK - 1)
            g = plsc.load_gather(buf, [pc])
            tl = jnp.where(inr, g, tl)
            plsc.store_scatter(buf, [pc], jnp.full((L,), NEG, jnp.float32),
                               mask=inr & (lane == 0))

            accs = (acc,) + tuple(
                jnp.full((L,), NEG, jnp.float32) for _ in range(UNROLL - 1))

            def scan_body(i, a, _buf=buf):
                o = i * (UNROLL * L)
                return tuple(
                    jnp.maximum(a[k], _buf[pl.ds(o + k * L, L)])
                    for k in range(UNROLL))

            accs = plsc.parallel_loop(0, VECS // UNROLL, 1,
                                      carry=accs)(scan_body)
            a = accs[0]
            for k in range(1, UNROLL):
                a = jnp.maximum(a, accs[k])
            acc = a
        target = jnp.max(acc)
        true_logit = jnp.max(tl)
        total = total + (target - true_logit)

    part_v[...] = jnp.broadcast_to(total, (L,))
    pltpu.sync_copy(part_v, out_hbm.at[wid])


_sc_kernel = functools.partial(
    pl.kernel,
    out_type=jax.ShapeDtypeStruct((NW, L), jnp.float32),
    mesh=plsc.VectorSubcoreMesh(core_axis_name="c", subcore_axis_name="s",
                                num_cores=NC, num_subcores=NS),
    compiler_params=pltpu.CompilerParams(needs_layout_passes=False,
                                         use_tc_tiling_on_sc=False),
    scratch_types=[
        pltpu.VMEM((ROWS,), jnp.int32),
        pltpu.VMEM((CHUNK,), jnp.float32),
        pltpu.VMEM((CHUNK,), jnp.float32),
        pltpu.VMEM((L,), jnp.float32),
        pltpu.SemaphoreType.DMA,
        pltpu.SemaphoreType.DMA,
    ],
)(_sc_body)


def _fin_body(x_ref, o_ref):
    o_ref[...] = jnp.sum(x_ref[:, 0:1]).reshape(1, 1)


def _finish(partials):
    return pl.pallas_call(
        _fin_body,
        out_shape=jax.ShapeDtypeStruct((1, 1), jnp.float32),
    )(partials)


@jax.jit
def kernel(pred, true):
    partials = _sc_kernel(pred, true.astype(jnp.int32))
    return _finish(partials)[0, 0]


# speedup vs baseline: 1.7268x; 1.7268x over previous
import functools

import jax
import jax.numpy as jnp
from jax import lax
from jax.experimental import pallas as pl
from jax.experimental.pallas import tpu as pltpu
from jax.experimental.pallas import tpu_sc as plsc

NC, NS, L = 2, 16, 16
NW = NC * NS
ROWS, COLS = 128, 100000
GR = 8
NG = ROWS // GR
CW = 6400
NT = CW // 128
CPW = 8
TAIL0 = (COLS // 128) * 128
TAILN = COLS - TAIL0
LASTW = TAIL0 - 15 * CW
NEG = float("-inf")


def _neg(n=L):
    return jnp.full((n,), NEG, jnp.float32)


def _splat_i32(x):
    return jnp.broadcast_to(x, (L,)).astype(jnp.int32)


def _sc_body(pred_hbm, true_hbm, out_hbm, true_v, buf0, buf1,
             acc_ref, tl_ref, part_v, sem0, sem1):
    core = lax.axis_index("c")
    s = lax.axis_index("s")
    g = core * (NS // 2) + s // 2
    h = s % 2
    row0 = pl.multiple_of(g * GR, GR)

    pltpu.sync_copy(true_hbm, true_v)

    bufs = (buf0, buf1)
    sems = (sem0, sem1)
    lane = lax.iota(jnp.int32, L)
    t_rs = [plsc.load_gather(true_v, [_splat_i32(row0 + r)])
            for r in range(GR)]

    def col0_of(j):
        return pl.multiple_of((2 * j + h) * CW, 128)

    def dma_args(j, bslot):
        if j < CPW - 1:
            return (pred_hbm.at[pl.ds(row0, GR), pl.ds(col0_of(j), CW)],
                    bufs[bslot], sems[bslot])
        return None

    def start_last(bslot):
        @pl.when(h == 0)
        def _():
            pltpu.async_copy(
                pred_hbm.at[pl.ds(row0, GR), pl.ds(14 * CW, CW)],
                bufs[bslot], sems[bslot])

        @pl.when(h == 1)
        def _():
            pltpu.async_copy(
                pred_hbm.at[pl.ds(row0, GR), pl.ds(15 * CW, LASTW)],
                bufs[bslot].at[:, pl.ds(0, LASTW)], sems[bslot])

    def scan_chunk(buf, accs, tls, buf_col0, nt):
        for r in range(GR):
            p = t_rs[r] - buf_col0
            inr = (p >= 0) & (p < nt * 128)
            pc = jnp.clip(p, 0, nt * 128 - 1)
            ridx = _splat_i32(r)
            gval = plsc.load_gather(buf, [ridx, pc])
            tls[r] = jnp.where(inr, gval, tls[r])
            plsc.store_scatter(buf, [ridx, pc], _neg(),
                               mask=inr & (lane == 0))
        for r in range(GR):
            def tile_body(t, a, _buf=buf, _r=r):
                loads = [_buf[_r, pl.ds(t * 128 + 16 * k, 16)]
                         for k in range(8)]
                m = jnp.maximum(
                    jnp.maximum(jnp.maximum(loads[0], loads[1]),
                                jnp.maximum(loads[2], loads[3])),
                    jnp.maximum(jnp.maximum(loads[4], loads[5]),
                                jnp.maximum(loads[6], loads[7])))
                return jnp.maximum(a, m)

            accs[r] = plsc.parallel_loop(0, nt, 1, carry=accs[r])(tile_body)
        return accs, tls

    accs = [_neg() for _ in range(GR)]
    tls = [_neg() for _ in range(GR)]

    descs = [None] * CPW
    descs[0] = pltpu.async_copy(*dma_args(0, 0))
    for j in range(CPW - 1):
        bslot = j % 2
        if j + 1 < CPW - 1:
            descs[j + 1] = pltpu.async_copy(*dma_args(j + 1, (j + 1) % 2))
        else:
            start_last((j + 1) % 2)
        descs[j].wait()
        accs, tls = scan_chunk(bufs[bslot], accs, tls, col0_of(j), NT)

    for r in range(GR):
        acc_ref[r, :] = accs[r]
        tl_ref[r, :] = tls[r]
    lslot = (CPW - 1) % 2

    @pl.when(h == 0)
    def _():
        pltpu.make_async_copy(
            pred_hbm.at[pl.ds(row0, GR), pl.ds(14 * CW, CW)],
            bufs[lslot], sems[lslot]).wait()
        a = [acc_ref[r, :] for r in range(GR)]
        t = [tl_ref[r, :] for r in range(GR)]
        a, t = scan_chunk(bufs[lslot], a, t, 14 * CW, NT)
        for r in range(GR):
            acc_ref[r, :] = a[r]
            tl_ref[r, :] = t[r]

    @pl.when(h == 1)
    def _():
        pltpu.make_async_copy(
            pred_hbm.at[pl.ds(row0, GR), pl.ds(15 * CW, LASTW)],
            bufs[lslot].at[:, pl.ds(0, LASTW)], sems[lslot]).wait()
        a = [acc_ref[r, :] for r in range(GR)]
        t = [tl_ref[r, :] for r in range(GR)]
        a, t = scan_chunk(bufs[lslot], a, t, 15 * CW, LASTW // 128)
        for r in range(GR):
            acc_ref[r, :] = a[r]
            tl_ref[r, :] = t[r]

    for r in range(GR):
        m_r = jnp.max(acc_ref[r, :])
        t_r = jnp.max(tl_ref[r, :])
        plsc.store_scatter(part_v, [_splat_i32(r), _splat_i32(0)],
                           jnp.broadcast_to(m_r, (L,)), mask=lane == 0)
        plsc.store_scatter(part_v, [_splat_i32(r), _splat_i32(1)],
                           jnp.broadcast_to(t_r, (L,)), mask=lane == 0)

    pltpu.sync_copy(part_v, out_hbm.at[h, pl.ds(row0, GR)])


_sc_kernel = functools.partial(
    pl.kernel,
    out_type=jax.ShapeDtypeStruct((2, ROWS, 2), jnp.float32),
    mesh=plsc.VectorSubcoreMesh(core_axis_name="c", subcore_axis_name="s",
                                num_cores=NC, num_subcores=NS),
    compiler_params=pltpu.CompilerParams(needs_layout_passes=False),
    scratch_types=[
        pltpu.VMEM((ROWS,), jnp.int32),
        pltpu.VMEM((GR, CW), jnp.float32),
        pltpu.VMEM((GR, CW), jnp.float32),
        pltpu.VMEM((GR, L), jnp.float32),
        pltpu.VMEM((GR, L), jnp.float32),
        pltpu.VMEM((GR, 2), jnp.float32),
        pltpu.SemaphoreType.DMA,
        pltpu.SemaphoreType.DMA,
    ],
)(_sc_body)


def _fin_body(p_ref, tail_ref, t_ref, o_ref):
    cid = lax.broadcasted_iota(jnp.int32, (ROWS, TAILN), 1) + TAIL0
    eq = cid == t_ref[...]
    tail = tail_ref[...]
    neg = jnp.full((ROWS, TAILN), NEG, jnp.float32)
    m_tail = jnp.max(jnp.where(eq, neg, tail), axis=1, keepdims=True)
    t_tail = jnp.max(jnp.where(eq, tail, neg), axis=1, keepdims=True)
    p = jnp.maximum(p_ref[0], p_ref[1])
    m = jnp.maximum(p[:, 0:1], m_tail)
    t = jnp.maximum(p[:, 1:2], t_tail)
    o_ref[...] = jnp.sum(m - t).reshape(1, 1)


def _finish(partials, tail, true2d):
    return pl.pallas_call(
        _fin_body,
        out_shape=jax.ShapeDtypeStruct((1, 1), jnp.float32),
    )(partials, tail, true2d)


@jax.jit
def kernel(pred, true):
    true32 = true.astype(jnp.int32)
    partials = _sc_kernel(pred, true32)
    tail = lax.slice(pred, (0, TAIL0), (ROWS, COLS))
    return _finish(partials, tail, true32.reshape(ROWS, 1))[0, 0]


# speedup vs baseline: 3.3775x; 1.9559x over previous
import functools

import jax
import jax.numpy as jnp
from jax import lax
from jax.experimental import pallas as pl
from jax.experimental.pallas import tpu as pltpu
from jax.experimental.pallas import tpu_sc as plsc

NC, NS, L = 2, 16, 16
NW = NC * NS
ROWS, COLS = 128, 100000
RG = ROWS // L
CH = 392
CPW = 8
SPAN = CH * CPW
STRIDE = 3128
LAST0 = COLS - SPAN
NEG = float("-inf")


def _splat_i32(x):
    return jnp.broadcast_to(x, (L,)).astype(jnp.int32)


def _sc_body(pred_hbm, true_hbm, out_hbm, true_v, buf0, buf1, part_v,
             sem0, sem1):
    core = lax.axis_index("c")
    s = lax.axis_index("s")
    w = core * NS + s
    base = pl.multiple_of(jnp.where(w < NW - 1, w * STRIDE, LAST0), 8)

    pltpu.sync_copy(true_hbm, true_v)

    bufs = (buf0, buf1)
    sems = (sem0, sem1)
    lane = lax.iota(jnp.int32, L)

    def dma(j, bslot):
        return pltpu.async_copy(
            pred_hbm.at[pl.ds(base + j * CH, CH), :], bufs[bslot],
            sems[bslot])

    accs = [jnp.full((L,), NEG, jnp.float32) for _ in range(RG)]
    tls = [jnp.full((L,), NEG, jnp.float32) for _ in range(RG)]
    tvs = [true_v[pl.ds(16 * j, L)] for j in range(RG)]

    descs = [None] * CPW
    descs[0] = dma(0, 0)
    for j in range(CPW):
        bslot = j % 2
        if j + 1 < CPW:
            descs[j + 1] = dma(j + 1, (j + 1) % 2)
        descs[j].wait()
        buf = bufs[bslot]
        c0 = base + j * CH

        for g in range(RG):
            p = tvs[g] - c0
            inr = (p >= 0) & (p < CH)
            pc = jnp.clip(p, 0, CH - 1)
            rows = lane + (16 * g)
            gval = plsc.load_gather(buf, [pc, rows], mask=inr)
            tls[g] = jnp.where(inr, gval, tls[g])
            plsc.store_scatter(buf, [pc, rows],
                               jnp.full((L,), NEG, jnp.float32), mask=inr)

        def class_body(i, a, _buf=buf):
            return tuple(
                jnp.maximum(a[k], _buf[i, pl.ds(16 * k, L)])
                for k in range(RG))

        accs = list(plsc.parallel_loop(0, CH, 1,
                                       carry=tuple(accs))(class_body))

    for g in range(RG):
        part_v[0, pl.ds(16 * g, L)] = accs[g]
        part_v[1, pl.ds(16 * g, L)] = tls[g]
    pltpu.sync_copy(part_v, out_hbm.at[w])


_sc_kernel = functools.partial(
    pl.kernel,
    out_type=jax.ShapeDtypeStruct((NW, 2, ROWS), jnp.float32),
    mesh=plsc.VectorSubcoreMesh(core_axis_name="c", subcore_axis_name="s",
                                num_cores=NC, num_subcores=NS),
    compiler_params=pltpu.CompilerParams(needs_layout_passes=False),
    scratch_types=[
        pltpu.VMEM((ROWS,), jnp.int32),
        pltpu.VMEM((CH, ROWS), jnp.float32),
        pltpu.VMEM((CH, ROWS), jnp.float32),
        pltpu.VMEM((2, ROWS), jnp.float32),
        pltpu.SemaphoreType.DMA,
        pltpu.SemaphoreType.DMA,
    ],
)(_sc_body)


def _fin_body(x_ref, o_ref):
    m = jnp.max(x_ref[:, 0, :], axis=0)
    t = jnp.max(x_ref[:, 1, :], axis=0)
    o_ref[...] = jnp.sum(m - t).reshape(1, 1)


def _finish(partials):
    return pl.pallas_call(
        _fin_body,
        out_shape=jax.ShapeDtypeStruct((1, 1), jnp.float32),
    )(partials)


@jax.jit
def kernel(pred, true):
    partials = _sc_kernel(pred.T, true.astype(jnp.int32))
    return _finish(partials)[0, 0]


# speedup vs baseline: 3.6064x; 1.0678x over previous
import functools

import jax
import jax.numpy as jnp
from jax import lax
from jax.experimental import pallas as pl
from jax.experimental.pallas import tpu as pltpu
from jax.experimental.pallas import tpu_sc as plsc

NC, NS, L = 2, 16, 16
NW = NC * NS
ROWS, COLS = 128, 100000
RG = ROWS // L
T0 = 56000
SC_N = COLS - T0
CH = 176
CPW = 8
SPAN = CH * CPW
STRIDE = 1376
LAST0 = COLS - SPAN
CHT = 2000
NEG = float("-inf")


def _splat_i32(x):
    return jnp.broadcast_to(x, (L,)).astype(jnp.int32)


def _sc_body(pred_hbm, true_hbm, out_hbm, true_v, buf0, buf1, part_v,
             sem0, sem1):
    core = lax.axis_index("c")
    s = lax.axis_index("s")
    w = core * NS + s
    base = pl.multiple_of(
        jnp.where(w < NW - 1, T0 + w * STRIDE, LAST0), 8)

    pltpu.sync_copy(true_hbm, true_v)

    bufs = (buf0, buf1)
    sems = (sem0, sem1)
    lane = lax.iota(jnp.int32, L)

    def dma(j, bslot):
        return pltpu.async_copy(
            pred_hbm.at[pl.ds(base + j * CH, CH), :], bufs[bslot],
            sems[bslot])

    accs = [jnp.full((L,), NEG, jnp.float32) for _ in range(RG)]
    tls = [jnp.full((L,), NEG, jnp.float32) for _ in range(RG)]
    tvs = [true_v[pl.ds(16 * j, L)] for j in range(RG)]

    descs = [None] * CPW
    descs[0] = dma(0, 0)
    for j in range(CPW):
        bslot = j % 2
        if j + 1 < CPW:
            descs[j + 1] = dma(j + 1, (j + 1) % 2)
        descs[j].wait()
        buf = bufs[bslot]
        c0 = base + j * CH

        for g in range(RG):
            p = tvs[g] - c0
            inr = (p >= 0) & (p < CH)
            pc = jnp.clip(p, 0, CH - 1)
            rows = lane + (16 * g)
            gval = plsc.load_gather(buf, [pc, rows], mask=inr)
            tls[g] = jnp.where(inr, gval, tls[g])
            plsc.store_scatter(buf, [pc, rows],
                               jnp.full((L,), NEG, jnp.float32), mask=inr)

        def class_body(i, a, _buf=buf):
            return tuple(
                jnp.maximum(a[k], _buf[i, pl.ds(16 * k, L)])
                for k in range(RG))

        accs = list(plsc.parallel_loop(0, CH, 1,
                                       carry=tuple(accs))(class_body))

    for g in range(RG):
        part_v[0, pl.ds(16 * g, L)] = accs[g]
        part_v[1, pl.ds(16 * g, L)] = tls[g]
    pltpu.sync_copy(part_v, out_hbm.at[w])


_sc_kernel = functools.partial(
    pl.kernel,
    out_type=jax.ShapeDtypeStruct((NW, 2, ROWS), jnp.float32),
    mesh=plsc.VectorSubcoreMesh(core_axis_name="c", subcore_axis_name="s",
                                num_cores=NC, num_subcores=NS),
    compiler_params=pltpu.CompilerParams(needs_layout_passes=False),
    scratch_types=[
        pltpu.VMEM((ROWS,), jnp.int32),
        pltpu.VMEM((CH, ROWS), jnp.float32),
        pltpu.VMEM((CH, ROWS), jnp.float32),
        pltpu.VMEM((2, ROWS), jnp.float32),
        pltpu.SemaphoreType.DMA,
        pltpu.SemaphoreType.DMA,
    ],
)(_sc_body)


def _tc_body(true_ref, x_ref, m_ref, t_ref):
    i = pl.program_id(0)
    ids = lax.broadcasted_iota(jnp.int32, (CHT, ROWS), 0) + i * CHT
    eq = ids == true_ref[...]
    x = x_ref[...]
    neg = jnp.full((CHT, ROWS), NEG, jnp.float32)
    mm = jnp.max(jnp.where(eq, neg, x), axis=0, keepdims=True)
    tt = jnp.max(jnp.where(eq, x, neg), axis=0, keepdims=True)

    @pl.when(i == 0)
    def _():
        m_ref[...] = mm
        t_ref[...] = tt

    @pl.when(i > 0)
    def _():
        m_ref[...] = jnp.maximum(m_ref[...], mm)
        t_ref[...] = jnp.maximum(t_ref[...], tt)


def _tc_head(predT, true2d):
    return pl.pallas_call(
        _tc_body,
        grid=(T0 // CHT,),
        in_specs=[
            pl.BlockSpec((1, ROWS), lambda i: (0, 0)),
            pl.BlockSpec((CHT, ROWS), lambda i: (i, 0)),
        ],
        out_specs=[
            pl.BlockSpec((1, ROWS), lambda i: (0, 0)),
            pl.BlockSpec((1, ROWS), lambda i: (0, 0)),
        ],
        out_shape=[jax.ShapeDtypeStruct((1, ROWS), jnp.float32)] * 2,
    )(true2d, predT)


def _fin_body(x_ref, m_ref, t_ref, o_ref):
    m = jnp.maximum(jnp.max(x_ref[:, 0, :], axis=0), m_ref[0, :])
    t = jnp.maximum(jnp.max(x_ref[:, 1, :], axis=0), t_ref[0, :])
    o_ref[...] = jnp.sum(m - t).reshape(1, 1)


def _finish(partials, tc_m, tc_t):
    return pl.pallas_call(
        _fin_body,
        out_shape=jax.ShapeDtypeStruct((1, 1), jnp.float32),
    )(partials, tc_m, tc_t)


@jax.jit
def kernel(pred, true):
    true32 = true.astype(jnp.int32)
    predT = pred.T
    partials = _sc_kernel(predT, true32)
    tc_m, tc_t = _tc_head(predT, true32.reshape(1, ROWS))
    return _finish(partials, tc_m, tc_t)[0, 0]


# speedup vs baseline: 3.9579x; 1.0975x over previous
import functools

import jax
import jax.numpy as jnp
from jax import lax
from jax.experimental import pallas as pl
from jax.experimental.pallas import tpu as pltpu
from jax.experimental.pallas import tpu_sc as plsc

NC, NS, L = 2, 16, 16
NW = NC * NS
ROWS, COLS = 128, 100000
RG = ROWS // L
RPW = ROWS // NW
T0 = 56000
CH = 344
CPW = 4
SPAN = CH * CPW
LAST0 = COLS - SPAN
CHT = 4000
NEG = float("-inf")


def _sc_body(pred_hbm, true_hbm, out_hbm, true_v, buf0, buf1, gbuf,
             part_v, sem0, sem1, semg):
    core = lax.axis_index("c")
    s = lax.axis_index("s")
    w = core * NS + s
    base = pl.multiple_of(
        jnp.where(w < NW - 1, T0 + w * SPAN, LAST0), 8)

    pltpu.sync_copy(true_hbm, true_v)
    lane = lax.iota(jnp.int32, L)

    lane4 = lane & 3
    rowsel = RPW * w + lane4
    tsel = plsc.load_gather(true_v, [rowsel])
    gdesc = pltpu.async_copy(pred_hbm.at[tsel], gbuf, semg)

    bufs = (buf0, buf1)
    sems = (sem0, sem1)

    def dma(j, bslot):
        return pltpu.async_copy(
            pred_hbm.at[pl.ds(base + j * CH, CH), :], bufs[bslot],
            sems[bslot])

    accs = [jnp.full((L,), NEG, jnp.float32) for _ in range(RG)]
    tvs = [true_v[pl.ds(16 * j, L)] for j in range(RG)]

    descs = [None] * CPW
    descs[0] = dma(0, 0)
    for j in range(CPW):
        bslot = j % 2
        if j + 1 < CPW:
            descs[j + 1] = dma(j + 1, (j + 1) % 2)
        descs[j].wait()
        buf = bufs[bslot]
        c0 = base + j * CH

        for g in range(RG):
            p = tvs[g] - c0
            inr = (p >= 0) & (p < CH)
            pc = jnp.clip(p, 0, CH - 1)
            plsc.store_scatter(buf, [pc, lane + (16 * g)],
                               jnp.full((L,), NEG, jnp.float32), mask=inr)

        def class_body(i, a, _buf=buf):
            return tuple(
                jnp.maximum(a[k], _buf[i, pl.ds(16 * k, L)])
                for k in range(RG))

        accs = list(plsc.parallel_loop(0, CH, 1,
                                       carry=tuple(accs))(class_body))

    for g in range(RG):
        part_v[0, pl.ds(16 * g, L)] = accs[g]
        part_v[1, pl.ds(16 * g, L)] = jnp.full((L,), NEG, jnp.float32)

    gdesc.wait()
    dval = plsc.load_gather(gbuf, [lane4, rowsel])
    plsc.store_scatter(part_v, [jnp.broadcast_to(1, (L,)).astype(jnp.int32),
                                rowsel], dval, mask=lane < RPW)

    pltpu.sync_copy(part_v, out_hbm.at[w])


_sc_kernel = functools.partial(
    pl.kernel,
    out_type=jax.ShapeDtypeStruct((NW, 2, ROWS), jnp.float32),
    mesh=plsc.VectorSubcoreMesh(core_axis_name="c", subcore_axis_name="s",
                                num_cores=NC, num_subcores=NS),
    compiler_params=pltpu.CompilerParams(needs_layout_passes=False),
    scratch_types=[
        pltpu.VMEM((ROWS,), jnp.int32),
        pltpu.VMEM((CH, ROWS), jnp.float32),
        pltpu.VMEM((CH, ROWS), jnp.float32),
        pltpu.VMEM((L, ROWS), jnp.float32),
        pltpu.VMEM((2, ROWS), jnp.float32),
        pltpu.SemaphoreType.DMA,
        pltpu.SemaphoreType.DMA,
        pltpu.SemaphoreType.DMA,
    ],
)(_sc_body)


def _tc_body(true_ref, x_ref, m_ref):
    i = pl.program_id(0)
    ids = lax.broadcasted_iota(jnp.int32, (CHT, ROWS), 0) + i * CHT
    eq = ids == true_ref[...]
    mm = jnp.max(jnp.where(eq, jnp.full((CHT, ROWS), NEG, jnp.float32),
                           x_ref[...]), axis=0, keepdims=True)

    @pl.when(i == 0)
    def _():
        m_ref[...] = mm

    @pl.when(i > 0)
    def _():
        m_ref[...] = jnp.maximum(m_ref[...], mm)


def _tc_head(predT, true2d):
    return pl.pallas_call(
        _tc_body,
        grid=(T0 // CHT,),
        in_specs=[
            pl.BlockSpec((1, ROWS), lambda i: (0, 0)),
            pl.BlockSpec((CHT, ROWS), lambda i: (i, 0)),
        ],
        out_specs=pl.BlockSpec((1, ROWS), lambda i: (0, 0)),
        out_shape=jax.ShapeDtypeStruct((1, ROWS), jnp.float32),
    )(true2d, predT)


def _fin_body(x_ref, m_ref, o_ref):
    m = jnp.maximum(jnp.max(x_ref[:, 0, :], axis=0), m_ref[0, :])
    t = jnp.max(x_ref[:, 1, :], axis=0)
    o_ref[...] = jnp.sum(m - t).reshape(1, 1)


def _finish(partials, tc_m):
    return pl.pallas_call(
        _fin_body,
        out_shape=jax.ShapeDtypeStruct((1, 1), jnp.float32),
    )(partials, tc_m)


@jax.jit
def kernel(pred, true):
    true32 = true.astype(jnp.int32)
    predT = pred.T
    partials = _sc_kernel(predT, true32)
    tc_m = _tc_head(predT, true32.reshape(1, ROWS))
    return _finish(partials, tc_m)[0, 0]
